# baseline (speedup 1.0000x reference)
import functools

import jax
import jax.numpy as jnp
from jax import lax
from jax.experimental import pallas as pl
from jax.experimental.pallas import tpu as pltpu
from jax.experimental.pallas import tpu_sc as plsc

C = 8192
F = 256
B = 16384
ALPHA = 0.1

NC = 2
NS = 16
FH = F // NC
RPT = B // NS
G = 128
NG = RPT // G
CPT = C // NS
CNTR = C // 128
ZR = 128
CC = 128
NCH = CPT // CC


def _sc_center(h, d2, center):
    mesh = plsc.VectorSubcoreMesh(
        core_axis_name="c", subcore_axis_name="s", num_cores=NC,
        num_subcores=NS)

    @functools.partial(
        pl.kernel,
        out_type=(
            jax.ShapeDtypeStruct((C, F), jnp.float32),
            jax.ShapeDtypeStruct((NC, NS, 16), jnp.float32),
            jax.ShapeDtypeStruct((NC, NS, 16), jnp.float32),
            jax.ShapeDtypeStruct((NC, NS, 16), jnp.float32),
        ),
        mesh=mesh,
        scratch_types=dict(
            seg_sh=pltpu.VMEM_SHARED((C, FH), jnp.float32),
            cnt_sh=pltpu.VMEM_SHARED((CNTR, 128), jnp.float32),
            zbuf=pltpu.VMEM((ZR, FH), jnp.float32),
            hbuf0=pltpu.VMEM((G, FH), jnp.float32),
            hbuf1=pltpu.VMEM((G, FH), jnp.float32),
            idxbuf=pltpu.VMEM((NG, G), jnp.int32),
            cntloc=pltpu.VMEM((CNTR, 128), jnp.float32),
            iotabuf=pltpu.VMEM((CNTR,), jnp.int32),
            facbuf=pltpu.VMEM((CPT,), jnp.float32),
            gbuf=pltpu.VMEM((CPT,), jnp.float32),
            cflat=pltpu.VMEM((CPT,), jnp.float32),
            pbuf=pltpu.VMEM((4, 16), jnp.float32),
            sem0=pltpu.SemaphoreType.DMA,
            sem1=pltpu.SemaphoreType.DMA,
            sem2=pltpu.SemaphoreType.DMA,
            sem3=pltpu.SemaphoreType.DMA,
            semo0=pltpu.SemaphoreType.DMA,
            semo1=pltpu.SemaphoreType.DMA,
        ),
        compiler_params=pltpu.CompilerParams(needs_layout_passes=False),
    )
    def k(h_hbm, d_hbm, c_hbm, nc_hbm, pdot_hbm, pcn_hbm, psq_hbm, seg_sh,
          cnt_sh, zbuf, hbuf0, hbuf1, idxbuf, cntloc, iotabuf, facbuf,
          gbuf, cflat, pbuf, sem0, sem1, sem2, sem3, semo0, semo1):
        cid = lax.axis_index("c")
        sid = lax.axis_index("s")
        bufs = (hbuf0, hbuf1)
        sems = (sem0, sem1)

        def h_load(g):
            row = sid * RPT + g * G
            return pltpu.async_copy(
                h_hbm.at[pl.ds(row, G), pl.ds(cid * FH, FH)],
                bufs[g % 2], sems[g % 2])

        cps = {0: h_load(0), 1: h_load(1)}
        idx_cp = pltpu.async_copy(d_hbm.at[pl.ds(sid * NG, NG)], idxbuf,
                                  sem2)

        def fill(i, _):
            for j in range(FH // 16):
                zbuf[i, pl.ds(j * 16, 16)] = jnp.zeros((16,), jnp.float32)
            return 0
        lax.fori_loop(0, ZR, fill, 0)

        zcps = [pltpu.async_copy(zbuf, seg_sh.at[pl.ds(sid * CPT + kk * ZR,
                                                       ZR)], sem3)
                for kk in range(CPT // ZR)]
        zcps.append(pltpu.async_copy(
            zbuf.at[pl.ds(0, CNTR // NS)],
            cnt_sh.at[pl.ds(sid * (CNTR // NS), CNTR // NS)], semo0))

        def zc(i, _):
            for j in range(8):
                cntloc[i, pl.ds(j * 16, 16)] = jnp.zeros(
                    (16,), jnp.float32)
            return 0
        lax.fori_loop(0, CNTR, zc, 0)
        for j in range(CNTR // 16):
            iotabuf[pl.ds(j * 16, 16)] = lax.iota(jnp.int32, 16) + j * 16

        idx_cp.wait()
        for cp in zcps:
            cp.wait()
        plsc.subcore_barrier()

        ones16 = jnp.ones((16,), jnp.float32)
        acc_h2 = jnp.zeros((16,), jnp.float32)
        for g in range(NG):
            cps[g].wait()
            buf = bufs[g % 2]
            scat = pltpu.async_copy(buf, seg_sh.at[idxbuf.at[g]],
                                    sem3, add=True)
            for j in range(G // 16):
                idx16 = idxbuf[g, pl.ds(j * 16, 16)]
                plsc.addupdate_scatter(cntloc, [idx16 >> 7, idx16 & 127],
                                       ones16)

            def sq(i, a, buf=buf):
                for j in range(FH // 16):
                    hv = buf[i, pl.ds(j * 16, 16)]
                    a = a + hv * hv
                return a
            acc_h2 = lax.fori_loop(0, G, sq, acc_h2)
            scat.wait()
            if g + 2 < NG:
                cps[g + 2] = h_load(g + 2)

        pltpu.sync_copy(cntloc, cnt_sh.at[iotabuf], add=True)

        plsc.subcore_barrier()

        pltpu.sync_copy(cnt_sh.at[pl.ds(sid * (CPT // 128), CPT // 128)],
                        cntloc.at[pl.ds(0, CPT // 128)])

        def mkfac(i, _):
            cv = cntloc[i // 8, pl.ds((i % 8) * 16, 16)]
            pos = cv > 0.0
            den = jnp.maximum(cv, 1.0)
            facbuf[pl.ds(i * 16, 16)] = jnp.where(pos, ALPHA / den, 0.0)
            gbuf[pl.ds(i * 16, 16)] = jnp.where(pos, 1.0 - ALPHA, 1.0)
            cflat[pl.ds(i * 16, 16)] = cv
            return 0
        lax.fori_loop(0, CPT // 16, mkfac, 0)

        ssems = (sem2, sem3)
        osems = (semo0, semo1)
        sslot = (hbuf1, zbuf)

        def c_load(cc):
            kbase = sid * CPT + cc * CC
            return pltpu.async_copy(
                c_hbm.at[pl.ds(kbase, CC), pl.ds(cid * FH, FH)], hbuf0,
                sem0)

        def s_load(cc):
            return pltpu.async_copy(
                seg_sh.at[pl.ds(sid * CPT + cc * CC, CC)],
                sslot[cc % 2], ssems[cc % 2])

        def nc_store(cc):
            kbase = sid * CPT + cc * CC
            return pltpu.async_copy(
                sslot[cc % 2],
                nc_hbm.at[pl.ds(kbase, CC), pl.ds(cid * FH, FH)],
                osems[cc % 2])

        acc_dot = jnp.zeros((16,), jnp.float32)
        acc_cn = jnp.zeros((16,), jnp.float32)
        cl = c_load(0)
        sl = {0: s_load(0), 1: s_load(1)}
        outs = {}
        for cc in range(NCH):
            cl.wait()
            sl[cc].wait()
            sref = sslot[cc % 2]

            def upd(bi, acc, cc=cc, sref=sref):
                ad, an = acc
                base = cc * CC + bi * 16
                fvec = facbuf[pl.ds(base, 16)]
                gvec = gbuf[pl.ds(base, 16)]
                nvec = cflat[pl.ds(base, 16)]
                for t in range(16):
                    i = bi * 16 + t
                    fk = fvec[t]
                    gk = gvec[t]
                    nk = nvec[t]
                    for j in range(FH // 16):
                        cv = hbuf0[i, pl.ds(j * 16, 16)]
                        sv = sref[i, pl.ds(j * 16, 16)]
                        sref[i, pl.ds(j * 16, 16)] = gk * cv + fk * sv
                        ad = ad + sv * cv
                        an = an + (nk * cv) * cv
                return (ad, an)
            acc_dot, acc_cn = lax.fori_loop(0, CC // 16, upd,
                                            (acc_dot, acc_cn))

            if cc + 1 < NCH:
                cl = c_load(cc + 1)
            if cc >= 1 and cc + 1 < NCH:
                outs.pop(cc - 1).wait()
                sl[cc + 1] = s_load(cc + 1)
            outs[cc] = nc_store(cc)
        for cc in sorted(outs):
            outs.pop(cc).wait()

        pbuf[0, pl.ds(0, 16)] = acc_dot
        pbuf[1, pl.ds(0, 16)] = acc_cn
        pbuf[2, pl.ds(0, 16)] = acc_h2
        pltpu.sync_copy(pbuf.at[0], pdot_hbm.at[cid, sid])
        pltpu.sync_copy(pbuf.at[1], pcn_hbm.at[cid, sid])
        pltpu.sync_copy(pbuf.at[2], psq_hbm.at[cid, sid])

    return k(h, d2, center)


def _tc_finish_body(pd_ref, pc_ref, ps_ref, loss_ref):
    loss = (jnp.sum(ps_ref[...]) - 2.0 * jnp.sum(pd_ref[...])
            + jnp.sum(pc_ref[...])) / (B * F)
    loss_ref[...] = loss.reshape(1, 1)


def _tc_finish(pdot, pcn, psq):
    return pl.pallas_call(
        _tc_finish_body,
        out_shape=jax.ShapeDtypeStruct((1, 1), jnp.float32),
    )(pdot, pcn, psq)


def kernel(h, d, center):
    d2 = d.astype(jnp.int32).reshape(B // 128, 128)
    new_center, pdot, pcn, psq = _sc_center(h, d2, center)
    loss2d = _tc_finish(pdot, pcn, psq)
    return loss2d[0, 0], new_center


# speedup vs baseline: 1.0295x; 1.0295x over previous
import functools

import jax
import jax.numpy as jnp
from jax import lax
from jax.experimental import pallas as pl
from jax.experimental.pallas import tpu as pltpu
from jax.experimental.pallas import tpu_sc as plsc

C = 8192
F = 256
B = 16384
ALPHA = 0.1

NC = 2
NS = 16
FH = F // NC
RPT = B // NS
G = 128
NG = RPT // G
CPT = C // NS
CNTR = C // 128
ZR = 128
CC = 128
NCH = CPT // CC


def _sc_center(h, d2, center):
    mesh = plsc.VectorSubcoreMesh(
        core_axis_name="c", subcore_axis_name="s", num_cores=NC,
        num_subcores=NS)

    @functools.partial(
        pl.kernel,
        out_type=(
            jax.ShapeDtypeStruct((C, F), jnp.float32),
            jax.ShapeDtypeStruct((NC, NS, 16), jnp.float32),
            jax.ShapeDtypeStruct((NC, NS, 16), jnp.float32),
        ),
        mesh=mesh,
        scratch_types=dict(
            seg_sh=pltpu.VMEM_SHARED((C, FH), jnp.float32),
            cnt_sh=pltpu.VMEM_SHARED((CNTR, 128), jnp.float32),
            zbuf=pltpu.VMEM((ZR, FH), jnp.float32),
            hbuf0=pltpu.VMEM((G, FH), jnp.float32),
            hbuf1=pltpu.VMEM((G, FH), jnp.float32),
            idxbuf=pltpu.VMEM((NG, G), jnp.int32),
            cntloc=pltpu.VMEM((CNTR, 128), jnp.float32),
            iotabuf=pltpu.VMEM((CNTR,), jnp.int32),
            facbuf=pltpu.VMEM((CPT,), jnp.float32),
            gbuf=pltpu.VMEM((CPT,), jnp.float32),
            cflat=pltpu.VMEM((CPT,), jnp.float32),
            pbuf=pltpu.VMEM((2, 16), jnp.float32),
            sem0=pltpu.SemaphoreType.DMA,
            sem1=pltpu.SemaphoreType.DMA,
            sem2=pltpu.SemaphoreType.DMA,
            sem3=pltpu.SemaphoreType.DMA,
            semo0=pltpu.SemaphoreType.DMA,
            semo1=pltpu.SemaphoreType.DMA,
        ),
        compiler_params=pltpu.CompilerParams(needs_layout_passes=False),
    )
    def k(h_hbm, d_hbm, c_hbm, nc_hbm, pdot_hbm, pcn_hbm, seg_sh, cnt_sh, zbuf,
          hbuf0, hbuf1, idxbuf, cntloc, iotabuf, facbuf, gbuf,
          cflat, pbuf, sem0, sem1, sem2, sem3, semo0, semo1):
        cid = lax.axis_index("c")
        sid = lax.axis_index("s")
        bufs = (hbuf0, hbuf1)
        sems = (sem0, sem1)

        def h_load(g):
            row = sid * RPT + g * G
            return pltpu.async_copy(
                h_hbm.at[pl.ds(row, G), pl.ds(cid * FH, FH)],
                bufs[g % 2], sems[g % 2])

        cps = {0: h_load(0), 1: h_load(1)}
        idx_cp = pltpu.async_copy(d_hbm.at[pl.ds(sid * NG, NG)], idxbuf,
                                  sem2)

        def fill(i, _):
            for j in range(FH // 16):
                zbuf[i, pl.ds(j * 16, 16)] = jnp.zeros((16,), jnp.float32)
            return 0
        lax.fori_loop(0, ZR, fill, 0)

        zcps = [pltpu.async_copy(zbuf, seg_sh.at[pl.ds(sid * CPT + kk * ZR,
                                                       ZR)], sem3)
                for kk in range(CPT // ZR)]
        zcps.append(pltpu.async_copy(
            zbuf.at[pl.ds(0, CNTR // NS)],
            cnt_sh.at[pl.ds(sid * (CNTR // NS), CNTR // NS)], semo0))

        def zc(i, _):
            for j in range(8):
                cntloc[i, pl.ds(j * 16, 16)] = jnp.zeros(
                    (16,), jnp.float32)
            return 0
        lax.fori_loop(0, CNTR, zc, 0)
        for j in range(CNTR // 16):
            iotabuf[pl.ds(j * 16, 16)] = lax.iota(jnp.int32, 16) + j * 16

        idx_cp.wait()
        for cp in zcps:
            cp.wait()
        plsc.subcore_barrier()

        ones16 = jnp.ones((16,), jnp.float32)
        for g in range(NG):
            cps[g].wait()
            scat = pltpu.async_copy(bufs[g % 2], seg_sh.at[idxbuf.at[g]],
                                    sem3, add=True)
            for j in range(G // 16):
                idx16 = idxbuf[g, pl.ds(j * 16, 16)]
                plsc.addupdate_scatter(cntloc, [idx16 >> 7, idx16 & 127],
                                       ones16)
            scat.wait()
            if g + 2 < NG:
                cps[g + 2] = h_load(g + 2)

        pltpu.sync_copy(cntloc, cnt_sh.at[iotabuf], add=True)

        plsc.subcore_barrier()

        pltpu.sync_copy(cnt_sh.at[pl.ds(sid * (CPT // 128), CPT // 128)],
                        cntloc.at[pl.ds(0, CPT // 128)])

        def mkfac(i, _):
            cv = cntloc[i // 8, pl.ds((i % 8) * 16, 16)]
            pos = cv > 0.0
            den = jnp.maximum(cv, 1.0)
            facbuf[pl.ds(i * 16, 16)] = jnp.where(pos, ALPHA / den, 0.0)
            gbuf[pl.ds(i * 16, 16)] = jnp.where(pos, 1.0 - ALPHA, 1.0)
            cflat[pl.ds(i * 16, 16)] = cv
            return 0
        lax.fori_loop(0, CPT // 16, mkfac, 0)

        ssems = (sem2, sem3)
        osems = (semo0, semo1)
        sslot = (hbuf1, zbuf)

        def c_load(cc):
            kbase = sid * CPT + cc * CC
            return pltpu.async_copy(
                c_hbm.at[pl.ds(kbase, CC), pl.ds(cid * FH, FH)], hbuf0,
                sem0)

        def s_load(cc):
            return pltpu.async_copy(
                seg_sh.at[pl.ds(sid * CPT + cc * CC, CC)],
                sslot[cc % 2], ssems[cc % 2])

        def nc_store(cc):
            kbase = sid * CPT + cc * CC
            return pltpu.async_copy(
                sslot[cc % 2],
                nc_hbm.at[pl.ds(kbase, CC), pl.ds(cid * FH, FH)],
                osems[cc % 2])

        acc_dot = jnp.zeros((16,), jnp.float32)
        acc_cn = jnp.zeros((16,), jnp.float32)
        cl = c_load(0)
        sl = {0: s_load(0), 1: s_load(1)}
        outs = {}
        for cc in range(NCH):
            cl.wait()
            sl[cc].wait()
            sref = sslot[cc % 2]

            def upd(bi, acc, cc=cc, sref=sref):
                ad, an = acc
                base = cc * CC + bi * 16
                fvec = facbuf[pl.ds(base, 16)]
                gvec = gbuf[pl.ds(base, 16)]
                nvec = cflat[pl.ds(base, 16)]
                for t in range(16):
                    i = bi * 16 + t
                    fk = fvec[t]
                    gk = gvec[t]
                    nk = nvec[t]
                    for j in range(FH // 16):
                        cv = hbuf0[i, pl.ds(j * 16, 16)]
                        sv = sref[i, pl.ds(j * 16, 16)]
                        sref[i, pl.ds(j * 16, 16)] = gk * cv + fk * sv
                        ad = ad + sv * cv
                        an = an + (nk * cv) * cv
                return (ad, an)
            acc_dot, acc_cn = lax.fori_loop(0, CC // 16, upd,
                                            (acc_dot, acc_cn))

            if cc + 1 < NCH:
                cl = c_load(cc + 1)
            if cc >= 1 and cc + 1 < NCH:
                outs.pop(cc - 1).wait()
                sl[cc + 1] = s_load(cc + 1)
            outs[cc] = nc_store(cc)
        for cc in sorted(outs):
            outs.pop(cc).wait()

        pbuf[0, pl.ds(0, 16)] = acc_dot
        pbuf[1, pl.ds(0, 16)] = acc_cn
        pltpu.sync_copy(pbuf.at[0], pdot_hbm.at[cid, sid])
        pltpu.sync_copy(pbuf.at[1], pcn_hbm.at[cid, sid])

    return k(h, d2, center)


HB = 2048


def _tc_sumsq_body(h_ref, o_ref):
    i = pl.program_id(0)
    hb = h_ref[...]

    @pl.when(i == 0)
    def _():
        o_ref[...] = jnp.zeros((1, 1), jnp.float32)

    o_ref[...] += jnp.sum(hb * hb).reshape(1, 1)


def _tc_sumsq(h):
    return pl.pallas_call(
        _tc_sumsq_body,
        grid=(B // HB,),
        in_specs=[pl.BlockSpec((HB, F), lambda i: (i, 0))],
        out_specs=pl.BlockSpec((1, 1), lambda i: (0, 0)),
        out_shape=jax.ShapeDtypeStruct((1, 1), jnp.float32),
    )(h)


def _tc_finish_body(s2_ref, pd_ref, pc_ref, loss_ref):
    loss = (s2_ref[0, 0] - 2.0 * jnp.sum(pd_ref[...])
            + jnp.sum(pc_ref[...])) / (B * F)
    loss_ref[...] = loss.reshape(1, 1)


def _tc_finish(sumh2, pdot, pcn):
    return pl.pallas_call(
        _tc_finish_body,
        out_shape=jax.ShapeDtypeStruct((1, 1), jnp.float32),
    )(sumh2, pdot, pcn)


def kernel(h, d, center):
    d2 = d.astype(jnp.int32).reshape(B // 128, 128)
    new_center, pdot, pcn = _sc_center(h, d2, center)
    sumh2 = _tc_sumsq(h)
    loss2d = _tc_finish(sumh2, pdot, pcn)
    return loss2d[0, 0], new_center
